# baseline (speedup 1.0000x reference)
import functools

import jax
import jax.numpy as jnp
from jax import lax
from jax.experimental import pallas as pl
from jax.experimental.pallas import tpu as pltpu
from jax.experimental.pallas import tpu_sc as plsc

_B = 4096
_D = 256
_NC = 2
_NS = 16
_L = 16
_NW = _NC * _NS
_BPW = _B // _NW
_CHUNKS = _D // _L
_SIZES = (16, 32, 32, 32, 16)
_OFFS = (0, 16, 48, 80, 112)
_NCH = len(_SIZES)

_mesh = plsc.VectorSubcoreMesh(core_axis_name="c", subcore_axis_name="s")


@functools.partial(
    pl.kernel,
    mesh=_mesh,
    out_type=jax.ShapeDtypeStruct((_NW, _L), jnp.float32),
    scratch_types=[
        pltpu.VMEM((_BPW,), jnp.int32),
        pltpu.VMEM((_BPW, _D), jnp.float32),
        pltpu.VMEM((_BPW, _D), jnp.float32),
        pltpu.VMEM((_L,), jnp.float32),
        pltpu.SemaphoreType.DMA,
        pltpu.SemaphoreType.DMA,
        pltpu.SemaphoreType.DMA,
        pltpu.SemaphoreType.DMA,
        pltpu.SemaphoreType.DMA,
    ],
)
def _center_loss_partials(feat_hbm, lab_hbm, cent_hbm, out_hbm,
                          idx_v, cent_v, feat_v, acc_v, s0, s1, s2, s3, s4):
    wid = lax.axis_index("s") * _NC + lax.axis_index("c")
    base = wid * _BPW
    sems = [s0, s1, s2, s3, s4]

    def issue_feat(j):
        return pltpu.async_copy(
            feat_hbm.at[pl.ds(base + _OFFS[j], _SIZES[j])],
            feat_v.at[pl.ds(_OFFS[j], _SIZES[j])], sems[j])

    def issue_gather(j):
        return pltpu.async_copy(
            cent_hbm.at[idx_v.at[pl.ds(_OFFS[j], _SIZES[j])]],
            cent_v.at[pl.ds(_OFFS[j], _SIZES[j])], sems[j])

    fc0 = issue_feat(0)
    pltpu.sync_copy(lab_hbm.at[pl.ds(base, _BPW)], idx_v)
    copies = [None] * _NCH
    copies[0] = (issue_gather(0), fc0)
    copies[1] = (issue_gather(1), issue_feat(1))

    def make_body(j):
        def body(r, accs):
            new = list(accs)
            for c in range(_CHUNKS):
                f = feat_v[_OFFS[j] + r, pl.ds(c * _L, _L)]
                g = cent_v[_OFFS[j] + r, pl.ds(c * _L, _L)]
                d = f - g
                new[c % 4] = new[c % 4] + d * d
            return tuple(new)
        return body

    zero = jnp.zeros((_L,), jnp.float32)
    accs = (zero, zero, zero, zero)
    for j in range(_NCH):
        copies[j][0].wait()
        copies[j][1].wait()
        if j + 2 < _NCH:
            copies[j + 2] = (issue_gather(j + 2), issue_feat(j + 2))
        accs = lax.fori_loop(0, _SIZES[j], make_body(j), accs)
    a0, a1, a2, a3 = accs
    acc_v[...] = (a0 + a1) + (a2 + a3)
    pltpu.sync_copy(acc_v, out_hbm.at[wid])


def kernel(features, labels, centers):
    partials = _center_loss_partials(features, labels.astype(jnp.int32), centers)
    return jnp.sum(partials) / jnp.float32(_B * _D)


# speedup vs baseline: 1.0182x; 1.0182x over previous
import functools

import jax
import jax.numpy as jnp
from jax import lax
from jax.experimental import pallas as pl
from jax.experimental.pallas import tpu as pltpu
from jax.experimental.pallas import tpu_sc as plsc

_B = 4096
_D = 256
_NC = 2
_NS = 16
_L = 16
_NW = _NC * _NS
_BPW = _B // _NW
_CHUNKS = _D // _L
_CH = 32
_NCH = _BPW // _CH

_mesh = plsc.VectorSubcoreMesh(core_axis_name="c", subcore_axis_name="s")


@functools.partial(
    pl.kernel,
    mesh=_mesh,
    out_type=jax.ShapeDtypeStruct((_NW, _L), jnp.float32),
    scratch_types=[
        pltpu.VMEM((_BPW,), jnp.int32),
        pltpu.VMEM((_NCH, _CH, _D), jnp.float32),
        pltpu.VMEM((_BPW, _D), jnp.float32),
        pltpu.VMEM((_L,), jnp.float32),
        pltpu.SemaphoreType.DMA,
        pltpu.SemaphoreType.DMA,
        pltpu.SemaphoreType.DMA,
        pltpu.SemaphoreType.DMA,
    ],
)
def _center_loss_partials(feat_hbm, lab_hbm, cent_hbm, out_hbm,
                          idx_v, cent_v, feat_v, acc_v, s0, s1, s2, s3):
    wid = lax.axis_index("s") * _NC + lax.axis_index("c")
    base = wid * _BPW
    sems = [s0, s1, s2, s3]

    def issue_feat(j):
        return pltpu.async_copy(feat_hbm.at[pl.ds(base + j * _CH, _CH)],
                                feat_v.at[pl.ds(j * _CH, _CH)], sems[j])

    def issue_gather(j):
        return pltpu.async_copy(cent_hbm.at[idx_v.at[pl.ds(j * _CH, _CH)]],
                                cent_v.at[j], sems[j])

    f0 = issue_feat(0)
    pltpu.sync_copy(lab_hbm.at[pl.ds(base, _BPW)], idx_v)
    copies = [None] * _NCH
    copies[0] = (issue_gather(0), f0)
    copies[1] = (issue_gather(1), issue_feat(1))

    def make_body(j):
        def body(r, accs):
            new = list(accs)
            for c in range(_CHUNKS):
                f = feat_v[j * _CH + r, pl.ds(c * _L, _L)]
                g = cent_v[j, r, pl.ds(c * _L, _L)]
                d = f - g
                new[c % 4] = new[c % 4] + d * d
            return tuple(new)
        return body

    zero = jnp.zeros((_L,), jnp.float32)
    accs = (zero, zero, zero, zero)
    for j in range(_NCH):
        copies[j][0].wait()
        copies[j][1].wait()
        if j + 2 < _NCH:
            copies[j + 2] = (issue_gather(j + 2), issue_feat(j + 2))
        accs = lax.fori_loop(0, _CH, make_body(j), accs)
    a0, a1, a2, a3 = accs
    acc_v[...] = (a0 + a1) + (a2 + a3)
    pltpu.sync_copy(acc_v, out_hbm.at[wid])


def kernel(features, labels, centers):
    partials = _center_loss_partials(features, labels.astype(jnp.int32), centers)
    return jnp.sum(partials) / jnp.float32(_B * _D)
